# baseline (speedup 1.0000x reference)
import functools

import jax
import jax.numpy as jnp
from jax import lax
from jax.experimental import pallas as pl
from jax.experimental.pallas import tpu as pltpu
from jax.experimental.pallas import tpu_sc as plsc

N = 10000
E = 320000
D = 128
H = 64
C = 10
G = 64

NC = 2
NS = 16
NW = NC * NS
EPW = E // NW
CH = 80
NCH = EPW // CH
NB = 6
NG = -(-NCH // NB)
RPS = 624
TAIL = N - NS * RPS

BLK = 5000
NBLK = N // BLK


def _edge_agg(y2, ei4, zeros):
    mesh = plsc.VectorSubcoreMesh(core_axis_name="c", subcore_axis_name="s")

    @functools.partial(
        pl.kernel,
        mesh=mesh,
        compiler_params=pltpu.CompilerParams(use_tc_tiling_on_sc=False),
        out_type=jax.ShapeDtypeStruct((N, 2 * H), jnp.float32),
        scratch_types=[
            pltpu.VMEM((NCH, CH), jnp.int32),
            pltpu.VMEM((NCH, CH), jnp.int32),
            pltpu.VMEM((NB, CH, H), jnp.float32),
            pltpu.VMEM_SHARED((N, H), jnp.float32),
            [pltpu.SemaphoreType.DMA] * NB,
        ],
    )
    def k(y_hbm, ei_hbm, z_hbm, out_hbm, src_v, dst_v, rows_v, acc, sems):
        c = lax.axis_index("c")
        s = lax.axis_index("s")
        wid = c * NS + s
        pltpu.sync_copy(z_hbm.at[pl.ds(s * RPS, RPS)], acc.at[pl.ds(s * RPS, RPS)])

        @pl.when(s == NS - 1)
        def _():
            pltpu.sync_copy(z_hbm.at[pl.ds(NS * RPS, TAIL)],
                            acc.at[pl.ds(NS * RPS, TAIL)])

        pltpu.sync_copy(ei_hbm.at[0, wid], src_v)
        pltpu.sync_copy(ei_hbm.at[1, wid], dst_v)
        plsc.subcore_barrier()

        def gather(j, b):
            pltpu.async_copy(
                y_hbm.at[src_v.at[j]], rows_v.at[b], sems[b]
            )

        for b in range(NB):
            gather(b, b)

        def body(g, carry):
            for b in range(NB):
                j = g * NB + b

                @pl.when(j < NCH)
                def _():
                    pltpu.make_async_copy(
                        y_hbm.at[src_v.at[j]], rows_v.at[b], sems[b]
                    ).wait()
                    pltpu.sync_copy(rows_v.at[b], acc.at[dst_v.at[j]],
                                    add=True)

                    @pl.when(j + NB < NCH)
                    def _():
                        gather(j + NB, b)

            return carry

        lax.fori_loop(0, NG, body, 0)
        plsc.subcore_barrier()
        pltpu.sync_copy(
            acc.at[pl.ds(s * RPS, RPS)],
            out_hbm.at[pl.ds(s * RPS, RPS), pl.ds(c * H, H)],
        )

        @pl.when(s == NS - 1)
        def _():
            pltpu.sync_copy(
                acc.at[pl.ds(NS * RPS, TAIL)],
                out_hbm.at[pl.ds(NS * RPS, TAIL), pl.ds(c * H, H)],
            )

    return k(y2, ei4, zeros)


def _mm_kernel(x_ref, w_ref, o_ref):
    y = jnp.dot(x_ref[...], w_ref[...], preferred_element_type=jnp.float32)
    o_ref[...] = jnp.concatenate([y, jnp.zeros_like(y)], axis=1)


def _first_matmul(x, w):
    return pl.pallas_call(
        _mm_kernel,
        grid=(NBLK,),
        in_specs=[
            pl.BlockSpec((BLK, D), lambda i: (i, 0)),
            pl.BlockSpec((D, H), lambda i: (0, 0)),
        ],
        out_specs=pl.BlockSpec((BLK, 2 * H), lambda i: (i, 0)),
        out_shape=jax.ShapeDtypeStruct((N, 2 * H), jnp.float32),
    )(x, w)


def _gin_tail(y_ref, p_ref, eps_ref, b1_ref, w2_ref, b2_ref):
    e = eps_ref[0, 0]
    p = p_ref[...]
    t = ((1.0 + e) * y_ref[...][:, :H] + p[:, :H] + p[:, H:] + b1_ref[...])
    t = jnp.maximum(t, 0.0)
    h = jnp.dot(t, w2_ref[...], preferred_element_type=jnp.float32) + b2_ref[...]
    return jnp.maximum(h, 0.0)


def _mid_kernel(y_ref, p_ref, eps_ref, b1_ref, w2_ref, b2_ref, wn_ref, o_ref):
    h = _gin_tail(y_ref, p_ref, eps_ref, b1_ref, w2_ref, b2_ref)
    yn = jnp.dot(h, wn_ref[...], preferred_element_type=jnp.float32)
    o_ref[...] = jnp.concatenate([yn, jnp.zeros_like(yn)], axis=1)


def _mid_stage(y2, parts, eps, b1, w2, b2, w_next):
    return pl.pallas_call(
        _mid_kernel,
        grid=(NBLK,),
        in_specs=[
            pl.BlockSpec((BLK, 2 * H), lambda i: (i, 0)),
            pl.BlockSpec((BLK, 2 * H), lambda i: (i, 0)),
            pl.BlockSpec((1, 1), lambda i: (0, 0)),
            pl.BlockSpec((1, H), lambda i: (0, 0)),
            pl.BlockSpec((H, H), lambda i: (0, 0)),
            pl.BlockSpec((1, H), lambda i: (0, 0)),
            pl.BlockSpec((H, H), lambda i: (0, 0)),
        ],
        out_specs=pl.BlockSpec((BLK, 2 * H), lambda i: (i, 0)),
        out_shape=jax.ShapeDtypeStruct((N, 2 * H), jnp.float32),
    )(y2, parts, eps.reshape(1, 1), b1.reshape(1, H), w2, b2.reshape(1, H),
      w_next)


def _final_kernel(y_ref, p_ref, eps_ref, b1_ref, w2_ref, b2_ref, batch_ref,
                  fcw_ref, fcb_ref, o_ref, pooled):
    i = pl.program_id(0)
    h = _gin_tail(y_ref, p_ref, eps_ref, b1_ref, w2_ref, b2_ref)
    oh = (lax.broadcasted_iota(jnp.int32, (G, BLK), 0) == batch_ref[0]
          ).astype(jnp.float32)
    blockpool = jnp.dot(oh, h, preferred_element_type=jnp.float32)

    @pl.when(i == 0)
    def _():
        pooled[...] = jnp.zeros_like(pooled)

    pooled[...] += blockpool

    @pl.when(i == NBLK - 1)
    def _():
        logits = jnp.dot(pooled[...], fcw_ref[...],
                         preferred_element_type=jnp.float32) + fcb_ref[...]
        m = jnp.max(logits, axis=1, keepdims=True)
        z = logits - m
        lse = jnp.log(jnp.sum(jnp.exp(z), axis=1, keepdims=True))
        o_ref[...] = z - lse


def _final_stage(y2, parts, eps, b1, w2, b2, batch, fc_w, fc_b):
    return pl.pallas_call(
        _final_kernel,
        grid=(NBLK,),
        in_specs=[
            pl.BlockSpec((BLK, 2 * H), lambda i: (i, 0)),
            pl.BlockSpec((BLK, 2 * H), lambda i: (i, 0)),
            pl.BlockSpec((1, 1), lambda i: (0, 0)),
            pl.BlockSpec((1, H), lambda i: (0, 0)),
            pl.BlockSpec((H, H), lambda i: (0, 0)),
            pl.BlockSpec((1, H), lambda i: (0, 0)),
            pl.BlockSpec((1, 1, BLK), lambda i: (i, 0, 0)),
            pl.BlockSpec((H, C), lambda i: (0, 0)),
            pl.BlockSpec((1, C), lambda i: (0, 0)),
        ],
        out_specs=pl.BlockSpec((G, C), lambda i: (0, 0)),
        out_shape=jax.ShapeDtypeStruct((G, C), jnp.float32),
        scratch_shapes=[pltpu.VMEM((G, H), jnp.float32)],
    )(y2, parts, eps.reshape(1, 1), b1.reshape(1, H), w2, b2.reshape(1, H),
      batch.reshape(NBLK, 1, BLK), fc_w, fc_b.reshape(1, C))


def kernel(x, edge_index, batch, l0_W1, l0_b1, l0_W2, l0_b2, eps0,
           l1_W1, l1_b1, l1_W2, l1_b2, eps1, fc_W, fc_b):
    ei4 = (edge_index * jnp.array([[2], [1]], jnp.int32)).reshape(
        2, NW, NCH, CH)
    zeros = jnp.zeros((N, H), jnp.float32)

    y0 = _first_matmul(x, l0_W1)
    parts0 = _edge_agg(y0.reshape(2 * N, H), ei4, zeros)
    y1 = _mid_stage(y0, parts0, eps0, l0_b1, l0_W2, l0_b2, l1_W1)
    parts1 = _edge_agg(y1.reshape(2 * N, H), ei4, zeros)
    return _final_stage(y1, parts1, eps1, l1_b1, l1_W2, l1_b2, batch,
                        fc_W, fc_b)


# speedup vs baseline: 1.0010x; 1.0010x over previous
import functools

import jax
import jax.numpy as jnp
from jax import lax
from jax.experimental import pallas as pl
from jax.experimental.pallas import tpu as pltpu
from jax.experimental.pallas import tpu_sc as plsc

N = 10000
E = 320000
D = 128
H = 64
C = 10
G = 64

NC = 2
NS = 16
NW = NC * NS
EPW = E // NW
CH = 80
NCH = EPW // CH
NB = 5
NG = -(-NCH // NB)
RPS = 624
TAIL = N - NS * RPS

BLK = 5000
NBLK = N // BLK


def _edge_agg(y2, ei4, zeros):
    mesh = plsc.VectorSubcoreMesh(core_axis_name="c", subcore_axis_name="s")

    @functools.partial(
        pl.kernel,
        mesh=mesh,
        compiler_params=pltpu.CompilerParams(use_tc_tiling_on_sc=False),
        out_type=jax.ShapeDtypeStruct((N, 2 * H), jnp.float32),
        scratch_types=[
            pltpu.VMEM((NCH, CH), jnp.int32),
            pltpu.VMEM((NCH, CH), jnp.int32),
            pltpu.VMEM((NB, CH, H), jnp.float32),
            pltpu.VMEM_SHARED((N, H), jnp.float32),
            [pltpu.SemaphoreType.DMA] * NB,
        ],
    )
    def k(y_hbm, ei_hbm, z_hbm, out_hbm, src_v, dst_v, rows_v, acc, sems):
        c = lax.axis_index("c")
        s = lax.axis_index("s")
        wid = c * NS + s
        pltpu.sync_copy(z_hbm.at[pl.ds(s * RPS, RPS)], acc.at[pl.ds(s * RPS, RPS)])

        @pl.when(s == NS - 1)
        def _():
            pltpu.sync_copy(z_hbm.at[pl.ds(NS * RPS, TAIL)],
                            acc.at[pl.ds(NS * RPS, TAIL)])

        pltpu.sync_copy(ei_hbm.at[0, wid], src_v)
        pltpu.sync_copy(ei_hbm.at[1, wid], dst_v)
        plsc.subcore_barrier()

        def gather(j, b):
            pltpu.async_copy(
                y_hbm.at[src_v.at[j]], rows_v.at[b], sems[b]
            )

        for b in range(NB):
            gather(b, b)

        def body(g, carry):
            for b in range(NB):
                j = g * NB + b

                @pl.when(j < NCH)
                def _():
                    pltpu.make_async_copy(
                        y_hbm.at[src_v.at[j]], rows_v.at[b], sems[b]
                    ).wait()
                    pltpu.sync_copy(rows_v.at[b], acc.at[dst_v.at[j]],
                                    add=True)

                    @pl.when(j + NB < NCH)
                    def _():
                        gather(j + NB, b)

            return carry

        lax.fori_loop(0, NG, body, 0)
        plsc.subcore_barrier()
        pltpu.sync_copy(
            acc.at[pl.ds(s * RPS, RPS)],
            out_hbm.at[pl.ds(s * RPS, RPS), pl.ds(c * H, H)],
        )

        @pl.when(s == NS - 1)
        def _():
            pltpu.sync_copy(
                acc.at[pl.ds(NS * RPS, TAIL)],
                out_hbm.at[pl.ds(NS * RPS, TAIL), pl.ds(c * H, H)],
            )

    return k(y2, ei4, zeros)


def _mm_kernel(x_ref, w_ref, o_ref):
    y = jnp.dot(x_ref[...], w_ref[...], preferred_element_type=jnp.float32)
    o_ref[...] = jnp.concatenate([y, jnp.zeros_like(y)], axis=1)


def _first_matmul(x, w):
    return pl.pallas_call(
        _mm_kernel,
        grid=(NBLK,),
        in_specs=[
            pl.BlockSpec((BLK, D), lambda i: (i, 0)),
            pl.BlockSpec((D, H), lambda i: (0, 0)),
        ],
        out_specs=pl.BlockSpec((BLK, 2 * H), lambda i: (i, 0)),
        out_shape=jax.ShapeDtypeStruct((N, 2 * H), jnp.float32),
    )(x, w)


def _gin_tail(y_ref, p_ref, eps_ref, b1_ref, w2_ref, b2_ref):
    e = eps_ref[0, 0]
    p = p_ref[...]
    t = ((1.0 + e) * y_ref[...][:, :H] + p[:, :H] + p[:, H:] + b1_ref[...])
    t = jnp.maximum(t, 0.0)
    h = jnp.dot(t, w2_ref[...], preferred_element_type=jnp.float32) + b2_ref[...]
    return jnp.maximum(h, 0.0)


def _mid_kernel(y_ref, p_ref, eps_ref, b1_ref, w2_ref, b2_ref, wn_ref, o_ref):
    h = _gin_tail(y_ref, p_ref, eps_ref, b1_ref, w2_ref, b2_ref)
    yn = jnp.dot(h, wn_ref[...], preferred_element_type=jnp.float32)
    o_ref[...] = jnp.concatenate([yn, jnp.zeros_like(yn)], axis=1)


def _mid_stage(y2, parts, eps, b1, w2, b2, w_next):
    return pl.pallas_call(
        _mid_kernel,
        grid=(NBLK,),
        in_specs=[
            pl.BlockSpec((BLK, 2 * H), lambda i: (i, 0)),
            pl.BlockSpec((BLK, 2 * H), lambda i: (i, 0)),
            pl.BlockSpec((1, 1), lambda i: (0, 0)),
            pl.BlockSpec((1, H), lambda i: (0, 0)),
            pl.BlockSpec((H, H), lambda i: (0, 0)),
            pl.BlockSpec((1, H), lambda i: (0, 0)),
            pl.BlockSpec((H, H), lambda i: (0, 0)),
        ],
        out_specs=pl.BlockSpec((BLK, 2 * H), lambda i: (i, 0)),
        out_shape=jax.ShapeDtypeStruct((N, 2 * H), jnp.float32),
    )(y2, parts, eps.reshape(1, 1), b1.reshape(1, H), w2, b2.reshape(1, H),
      w_next)


def _final_kernel(y_ref, p_ref, eps_ref, b1_ref, w2_ref, b2_ref, batch_ref,
                  fcw_ref, fcb_ref, o_ref, pooled):
    i = pl.program_id(0)
    h = _gin_tail(y_ref, p_ref, eps_ref, b1_ref, w2_ref, b2_ref)
    oh = (lax.broadcasted_iota(jnp.int32, (G, BLK), 0) == batch_ref[0]
          ).astype(jnp.float32)
    blockpool = jnp.dot(oh, h, preferred_element_type=jnp.float32)

    @pl.when(i == 0)
    def _():
        pooled[...] = jnp.zeros_like(pooled)

    pooled[...] += blockpool

    @pl.when(i == NBLK - 1)
    def _():
        logits = jnp.dot(pooled[...], fcw_ref[...],
                         preferred_element_type=jnp.float32) + fcb_ref[...]
        m = jnp.max(logits, axis=1, keepdims=True)
        z = logits - m
        lse = jnp.log(jnp.sum(jnp.exp(z), axis=1, keepdims=True))
        o_ref[...] = z - lse


def _final_stage(y2, parts, eps, b1, w2, b2, batch, fc_w, fc_b):
    return pl.pallas_call(
        _final_kernel,
        grid=(NBLK,),
        in_specs=[
            pl.BlockSpec((BLK, 2 * H), lambda i: (i, 0)),
            pl.BlockSpec((BLK, 2 * H), lambda i: (i, 0)),
            pl.BlockSpec((1, 1), lambda i: (0, 0)),
            pl.BlockSpec((1, H), lambda i: (0, 0)),
            pl.BlockSpec((H, H), lambda i: (0, 0)),
            pl.BlockSpec((1, H), lambda i: (0, 0)),
            pl.BlockSpec((1, 1, BLK), lambda i: (i, 0, 0)),
            pl.BlockSpec((H, C), lambda i: (0, 0)),
            pl.BlockSpec((1, C), lambda i: (0, 0)),
        ],
        out_specs=pl.BlockSpec((G, C), lambda i: (0, 0)),
        out_shape=jax.ShapeDtypeStruct((G, C), jnp.float32),
        scratch_shapes=[pltpu.VMEM((G, H), jnp.float32)],
    )(y2, parts, eps.reshape(1, 1), b1.reshape(1, H), w2, b2.reshape(1, H),
      batch.reshape(NBLK, 1, BLK), fc_w, fc_b.reshape(1, C))


def kernel(x, edge_index, batch, l0_W1, l0_b1, l0_W2, l0_b2, eps0,
           l1_W1, l1_b1, l1_W2, l1_b2, eps1, fc_W, fc_b):
    ei4 = (edge_index * jnp.array([[2], [1]], jnp.int32)).reshape(
        2, NW, NCH, CH)
    zeros = jnp.zeros((N, H), jnp.float32)

    y0 = _first_matmul(x, l0_W1)
    parts0 = _edge_agg(y0.reshape(2 * N, H), ei4, zeros)
    y1 = _mid_stage(y0, parts0, eps0, l0_b1, l0_W2, l0_b2, l1_W1)
    parts1 = _edge_agg(y1.reshape(2 * N, H), ei4, zeros)
    return _final_stage(y1, parts1, eps1, l1_b1, l1_W2, l1_b2, batch,
                        fc_W, fc_b)
